# speedup vs baseline: 11.8081x; 11.8081x over previous
import functools

import jax
import jax.numpy as jnp
from jax import lax
from jax.experimental import pallas as pl
from jax.experimental.pallas import tpu as pltpu
from jax.experimental.pallas import tpu_sc as plsc

N_NODES = 10000
D = 128
N_PAD = 10240
NC, NS = 2, 16
NW = NC * NS
EB = 128
ROWS_PER_TILE = N_PAD // NS


def _sc_degree(dst_w, zeros_1d, ones_eb):
  ch = dst_w.shape[1]

  @functools.partial(
      pl.kernel,
      out_type=jax.ShapeDtypeStruct((NC, N_PAD), jnp.float32),
      mesh=plsc.VectorSubcoreMesh(core_axis_name="c", subcore_axis_name="s"),
      scratch_types=[
          pltpu.VMEM((ch, EB), jnp.int32),
          pltpu.VMEM((EB,), jnp.float32),
          pltpu.VMEM_SHARED((N_PAD,), jnp.float32),
      ],
  )
  def k(dst_hbm, z_hbm, ones_hbm, out_hbm, dst_v, ones_v, acc_sh):
    c = lax.axis_index("c")
    s = lax.axis_index("s")
    wid = c * NS + s
    pltpu.sync_copy(z_hbm, acc_sh.at[pl.ds(s * ROWS_PER_TILE, ROWS_PER_TILE)])
    pltpu.sync_copy(dst_hbm.at[wid], dst_v)
    pltpu.sync_copy(ones_hbm, ones_v)
    plsc.subcore_barrier()

    def body(j, carry):
      pltpu.sync_copy(ones_v, acc_sh.at[dst_v.at[j]], add=True)
      return carry

    lax.fori_loop(0, ch, body, 0)
    plsc.subcore_barrier()
    pltpu.sync_copy(acc_sh.at[pl.ds(s * ROWS_PER_TILE, ROWS_PER_TILE)],
                    out_hbm.at[c].at[pl.ds(s * ROWS_PER_TILE, ROWS_PER_TILE)])

  return k(dst_w, zeros_1d, ones_eb)


def _sc_scatter(g, src_w, dst_w, zeros_2d):
  ch = src_w.shape[1]

  @functools.partial(
      pl.kernel,
      out_type=jax.ShapeDtypeStruct((NC, N_PAD, D), jnp.float32),
      mesh=plsc.VectorSubcoreMesh(core_axis_name="c", subcore_axis_name="s"),
      scratch_types=[
          pltpu.VMEM((ch, EB), jnp.int32),
          pltpu.VMEM((ch, EB), jnp.int32),
          pltpu.VMEM((EB, D), jnp.float32),
          pltpu.VMEM_SHARED((N_PAD, D), jnp.float32),
          pltpu.SemaphoreType.DMA,
      ],
  )
  def k(g_hbm, src_hbm, dst_hbm, z_hbm, out_hbm, src_v, dst_v, rows_v, acc_sh,
        sem):
    c = lax.axis_index("c")
    s = lax.axis_index("s")
    wid = c * NS + s
    pltpu.sync_copy(z_hbm, acc_sh.at[pl.ds(s * ROWS_PER_TILE, ROWS_PER_TILE)])
    pltpu.sync_copy(src_hbm.at[wid], src_v)
    pltpu.sync_copy(dst_hbm.at[wid], dst_v)
    plsc.subcore_barrier()

    def body(j, carry):
      pltpu.async_copy(g_hbm.at[src_v.at[j]], rows_v, sem).wait()
      pltpu.sync_copy(rows_v, acc_sh.at[dst_v.at[j]], add=True)
      return carry

    lax.fori_loop(0, ch, body, 0)
    plsc.subcore_barrier()
    pltpu.sync_copy(acc_sh.at[pl.ds(s * ROWS_PER_TILE, ROWS_PER_TILE)],
                    out_hbm.at[c].at[pl.ds(s * ROWS_PER_TILE, ROWS_PER_TILE)])

  return k(g, src_w, dst_w, zeros_2d)


_BLK = 1024
_GRID = N_PAD // _BLK


def _dis_body(p_ref, o_ref):
  o_ref[:] = lax.rsqrt(p_ref[0] + p_ref[1] + 1.0)


def _tc_dis(deg_parts):
  return pl.pallas_call(
      _dis_body,
      out_shape=jax.ShapeDtypeStruct((N_PAD // 128, 128), jnp.float32),
  )(deg_parts)


def _g0_body(x_ref, w_ref, dis_ref, o_ref):
  m = jnp.dot(x_ref[:], w_ref[:], preferred_element_type=jnp.float32)
  o_ref[:] = m * dis_ref[:]


def _tc_g0(x_pad, w0, dis_col):
  return pl.pallas_call(
      _g0_body,
      grid=(_GRID,),
      in_specs=[
          pl.BlockSpec((_BLK, D), lambda i: (i, 0)),
          pl.BlockSpec((D, D), lambda i: (0, 0)),
          pl.BlockSpec((_BLK, 1), lambda i: (i, 0)),
      ],
      out_specs=pl.BlockSpec((_BLK, D), lambda i: (i, 0)),
      out_shape=jax.ShapeDtypeStruct((N_PAD, D), jnp.float32),
  )(x_pad, w0, dis_col)


def _mid_body(s_ref, g_ref, dis_ref, b0_ref, gam_ref, bet_ref, w1_ref, o_ref):
  dis = dis_ref[:]
  t = (s_ref[0] + s_ref[1] + g_ref[:]) * dis + b0_ref[:]
  mu = jnp.mean(t, axis=1, keepdims=True)
  var = jnp.mean((t - mu) * (t - mu), axis=1, keepdims=True)
  h = (t - mu) * lax.rsqrt(var + 1e-5) * gam_ref[:] + bet_ref[:]
  h = jnp.maximum(h, 0.0)
  o_ref[:] = jnp.dot(h, w1_ref[:], preferred_element_type=jnp.float32) * dis


def _tc_mid(s0, g0, dis_col, b0, gamma, beta, w1):
  return pl.pallas_call(
      _mid_body,
      grid=(_GRID,),
      in_specs=[
          pl.BlockSpec((NC, _BLK, D), lambda i: (0, i, 0)),
          pl.BlockSpec((_BLK, D), lambda i: (i, 0)),
          pl.BlockSpec((_BLK, 1), lambda i: (i, 0)),
          pl.BlockSpec((1, D), lambda i: (0, 0)),
          pl.BlockSpec((1, D), lambda i: (0, 0)),
          pl.BlockSpec((1, D), lambda i: (0, 0)),
          pl.BlockSpec((D, D), lambda i: (0, 0)),
      ],
      out_specs=pl.BlockSpec((_BLK, D), lambda i: (i, 0)),
      out_shape=jax.ShapeDtypeStruct((N_PAD, D), jnp.float32),
  )(s0, g0, dis_col, b0, gamma, beta, w1)


def _fin_body(s_ref, g_ref, dis_ref, b1_ref, o_ref):
  o_ref[:] = (s_ref[0] + s_ref[1] + g_ref[:]) * dis_ref[:] + b1_ref[:]


def _tc_fin(s1, g1, dis_col, b1):
  return pl.pallas_call(
      _fin_body,
      grid=(_GRID,),
      in_specs=[
          pl.BlockSpec((NC, _BLK, D), lambda i: (0, i, 0)),
          pl.BlockSpec((_BLK, D), lambda i: (i, 0)),
          pl.BlockSpec((_BLK, 1), lambda i: (i, 0)),
          pl.BlockSpec((1, D), lambda i: (0, 0)),
      ],
      out_specs=pl.BlockSpec((_BLK, D), lambda i: (i, 0)),
      out_shape=jax.ShapeDtypeStruct((N_PAD, D), jnp.float32),
  )(s1, g1, dis_col, b1)


def kernel(x, edge_index, W0, b0, gamma, beta, W1, b1):
  n, d = x.shape
  e = edge_index.shape[1]
  ch = -(-e // (NW * EB))
  e_pad = NW * ch * EB

  src = edge_index[0].astype(jnp.int32)
  dst = edge_index[1].astype(jnp.int32)
  pad = e_pad - e
  src_w = jnp.concatenate(
      [src, jnp.full((pad,), N_NODES, jnp.int32)]).reshape(NW, ch, EB)
  dst_w = jnp.concatenate(
      [dst, jnp.full((pad,), N_NODES, jnp.int32)]).reshape(NW, ch, EB)

  x_pad = jnp.zeros((N_PAD, d), x.dtype).at[:n].set(x)
  zeros_1d = jnp.zeros((ROWS_PER_TILE,), jnp.float32)
  zeros_2d = jnp.zeros((ROWS_PER_TILE, D), jnp.float32)
  ones_eb = jnp.ones((EB,), jnp.float32)
  b0r = b0.reshape(1, D)
  b1r = b1.reshape(1, D)
  gammar = gamma.reshape(1, D)
  betar = beta.reshape(1, D)

  deg_parts = _sc_degree(dst_w, zeros_1d, ones_eb)
  dis2d = _tc_dis(deg_parts.reshape(NC, N_PAD // 128, 128))
  dis_col = dis2d.reshape(N_PAD, 1)

  g0 = _tc_g0(x_pad, W0, dis_col)
  s0 = _sc_scatter(g0, src_w, dst_w, zeros_2d)
  g1 = _tc_mid(s0, g0, dis_col, b0r, gammar, betar, W1)
  s1 = _sc_scatter(g1, src_w, dst_w, zeros_2d)
  out = _tc_fin(s1, g1, dis_col, b1r)
  return out[:n]
